# baseline (speedup 1.0000x reference)
import functools

import jax
import jax.numpy as jnp
from jax import lax
from jax.experimental import pallas as pl
from jax.experimental.pallas import tpu as pltpu
from jax.experimental.pallas import tpu_sc as plsc

N = 10000
E = 320000
D = 128
H1 = 50
DE = 16
H2 = 80
TW = 64

CH = 128
NCHUNK_REAL = E // CH
NC, NS = 2, 16
NTILE = NC * NS
CPT = 80
NBUF = 6
assert (NTILE - 1) * CPT < NCHUNK_REAL <= NTILE * CPT
RPT = 632
N_ACC = RPT * NS

ROWS_BLK = 2000
NBLK = N // ROWS_BLK

_SELU_ALPHA = 1.6732632423543772
_SELU_SCALE = 1.0507009873554805


def _selu(x):
    return _SELU_SCALE * jnp.where(x > 0, x, _SELU_ALPHA * (jnp.exp(x) - 1.0))


def _mesh():
    return plsc.VectorSubcoreMesh(
        core_axis_name="c", subcore_axis_name="s",
        num_cores=NC, num_subcores=NS)


TAIL = NCHUNK_REAL - (NTILE - 1) * CPT


def _stage_idx(idx3, buf, base, full):
    @pl.when(full)
    def _():
        pltpu.sync_copy(idx3.at[pl.ds(base, CPT)], buf)

    @pl.when(jnp.logical_not(full))
    def _():
        pltpu.sync_copy(idx3.at[pl.ds(base, TAIL)], buf.at[pl.ds(0, TAIL)])


@functools.lru_cache(maxsize=None)
def _make_edge_agg():
    scratch = [
        pltpu.VMEM_SHARED((N_ACC, TW), jnp.float32),
        pltpu.VMEM((CPT, 2, CH), jnp.int32),
        pltpu.VMEM((NBUF, CH, TW), jnp.float32),
        pltpu.SemaphoreType.DMA((NBUF,)),
        pltpu.SemaphoreType.DMA((NBUF,)),
    ]

    def body(table, idx3, z64, out_agg, acc, idxl, rows, gsem, ssem):
        c = lax.axis_index("c")
        s = lax.axis_index("s")
        base = (c * NS + s) * CPT

        pltpu.sync_copy(z64.at[pl.ds(s * RPT, RPT)], acc.at[pl.ds(s * RPT, RPT)])

        n_real = jnp.minimum(NCHUNK_REAL - base, CPT)
        _stage_idx(idx3, idxl, base, base + CPT <= NCHUNK_REAL)

        plsc.subcore_barrier()

        def fire_g(j, slot):
            pltpu.async_copy(table.at[idxl.at[j, 0]], rows.at[slot],
                             gsem.at[slot])

        def wait_g(j, slot):
            pltpu.make_async_copy(table.at[idxl.at[j, 0]], rows.at[slot],
                                  gsem.at[slot]).wait()

        def fire_s(j, slot):
            pltpu.async_copy(rows.at[slot], acc.at[idxl.at[j, 1]],
                             ssem.at[slot], add=True)

        def wait_s(j, slot):
            pltpu.make_async_copy(rows.at[slot], acc.at[idxl.at[j, 1]],
                                  ssem.at[slot]).wait()

        for p in range(NBUF - 1):
            @pl.when(p < n_real)
            def _(p=p):
                fire_g(p, p)

        def step(j, carry):
            slot = lax.rem(j, NBUF)
            fslot = lax.rem(j + NBUF - 1, NBUF)

            @pl.when(j + NBUF - 1 < n_real)
            def _():
                @pl.when(j >= 1)
                def _():
                    wait_s(j - 1, fslot)

                fire_g(j + NBUF - 1, fslot)

            wait_g(j, slot)
            fire_s(j, slot)
            return carry

        lax.fori_loop(0, n_real, step, 0)

        def drain(k, carry):
            j = n_real - NBUF + k

            @pl.when(j >= 0)
            def _():
                wait_s(j, lax.rem(j, NBUF))
            return carry

        lax.fori_loop(0, NBUF, drain, 0)

        plsc.subcore_barrier()

        pltpu.sync_copy(acc.at[pl.ds(s * RPT, RPT)],
                        out_agg.at[c, pl.ds(s * RPT, RPT)])

    return pl.kernel(body,
                     out_type=jax.ShapeDtypeStruct((NC, N_ACC, TW),
                                                   jnp.float32),
                     mesh=_mesh(), scratch_types=scratch,
                     compiler_params=pltpu.CompilerParams(
                         use_tc_tiling_on_sc=False))


@functools.lru_cache(maxsize=None)
def _make_attr_agg():
    scratch = [
        pltpu.VMEM_SHARED((N_ACC, DE), jnp.float32),
        pltpu.VMEM((CPT, 2, CH), jnp.int32),
        pltpu.VMEM((2, CH, DE), jnp.float32),
        pltpu.SemaphoreType.DMA((2,)),
        pltpu.SemaphoreType.DMA((2,)),
    ]

    def body(idx3, attr, z16, out_sattr, acc2, idxl, attrb, asem, ssem):
        c = lax.axis_index("c")
        s = lax.axis_index("s")
        base = (c * NS + s) * CPT

        pltpu.sync_copy(z16.at[pl.ds(s * RPT, RPT)],
                        acc2.at[pl.ds(s * RPT, RPT)])

        n_real = jnp.minimum(NCHUNK_REAL - base, CPT)
        _stage_idx(idx3, idxl, base, base + CPT <= NCHUNK_REAL)

        plsc.subcore_barrier()

        def fire_a(j, slot):
            pltpu.async_copy(attr.at[pl.ds((base + j) * CH, CH)],
                             attrb.at[slot], asem.at[slot])

        def wait_a(j, slot):
            pltpu.make_async_copy(attr.at[pl.ds((base + j) * CH, CH)],
                                  attrb.at[slot], asem.at[slot]).wait()

        def fire_s(j, slot):
            pltpu.async_copy(attrb.at[slot], acc2.at[idxl.at[j, 1]],
                             ssem.at[slot], add=True)

        def wait_s(j, slot):
            pltpu.make_async_copy(attrb.at[slot], acc2.at[idxl.at[j, 1]],
                                  ssem.at[slot]).wait()

        fire_a(0, 0)

        def step(j, carry):
            slot = lax.rem(j, 2)
            nslot = lax.rem(j + 1, 2)

            @pl.when(j + 1 < n_real)
            def _():
                @pl.when(j >= 1)
                def _():
                    wait_s(j - 1, nslot)

                fire_a(j + 1, nslot)

            wait_a(j, slot)
            fire_s(j, slot)
            return carry

        lax.fori_loop(0, n_real, step, 0)

        wait_s(n_real - 2, lax.rem(n_real - 2, 2))
        wait_s(n_real - 1, lax.rem(n_real - 1, 2))

        plsc.subcore_barrier()

        pltpu.sync_copy(acc2.at[pl.ds(s * RPT, RPT)],
                        out_sattr.at[c, pl.ds(s * RPT, RPT)])

    return pl.kernel(body,
                     out_type=jax.ShapeDtypeStruct((NC, N_ACC, DE),
                                                   jnp.float32),
                     mesh=_mesh(), scratch_types=scratch,
                     compiler_params=pltpu.CompilerParams(
                         use_tc_tiling_on_sc=False))


def _dot(a, b):
    return jnp.dot(a, b, preferred_element_type=jnp.float32)


def _pre0_body(x_ref, vw_ref, oh_ref, out_ref):
    out_ref[...] = _dot(x_ref[...], vw_ref[...]) + oh_ref[...]


def _mid_body(h_ref, agg_ref, sat_ref, a_ref, b_ref, dm_ref, bias_ref,
              vw_ref, oh_ref, h_out, hv_out):
    agg = agg_ref[0] + agg_ref[1]
    sat = sat_ref[0] + sat_ref[1]
    z = (_dot(h_ref[...], a_ref[...]) + _dot(agg, b_ref[...])
         + _dot(sat, dm_ref[...]) + bias_ref[...])
    h = _selu(z)
    h_out[...] = h
    hv_out[...] = _dot(h, vw_ref[...]) + oh_ref[...]


def _final_body(h_ref, agg_ref, sat_ref, x_ref, a_ref, b_ref, dm_ref,
                bias_ref, rh_ref, rx_ref, rb_ref, out_ref):
    i = pl.program_id(0)
    agg = agg_ref[0] + agg_ref[1]
    sat = sat_ref[0] + sat_ref[1]
    z = (_dot(h_ref[...], a_ref[...]) + _dot(agg, b_ref[...])
         + _dot(sat, dm_ref[...]) + bias_ref[...])
    h = _selu(z)
    r = _selu(_dot(h, rh_ref[...]) + _dot(x_ref[...], rx_ref[...])
              + rb_ref[...])

    @pl.when(i == 0)
    def _():
        out_ref[...] = jnp.zeros_like(out_ref)

    out_ref[...] += jnp.sum(r, axis=0, keepdims=True)


def _rows(shape):
    return pl.BlockSpec(shape, lambda i: (i, 0))


def _whole2(shape):
    return pl.BlockSpec(shape, lambda i: (0, 0))


def _whole3(shape):
    return pl.BlockSpec(shape, lambda i: (0, i, 0))


def _pre0(x, vw, oh):
    return pl.pallas_call(
        _pre0_body, grid=(NBLK,),
        in_specs=[_rows((ROWS_BLK, D)), _whole2((D, TW)), _whole2((1, TW))],
        out_specs=_rows((ROWS_BLK, TW)),
        out_shape=jax.ShapeDtypeStruct((N, TW), jnp.float32),
    )(x, vw, oh)


def _mid(h, agg_p, sat_p, a, b, dm, bias, vw, oh):
    return pl.pallas_call(
        _mid_body, grid=(NBLK,),
        in_specs=[
            _rows((ROWS_BLK, D)),
            _whole3((NC, ROWS_BLK, TW)),
            _whole3((NC, ROWS_BLK, DE)),
            _whole2((D, D)), _whole2((TW, D)), _whole2((DE, D)),
            _whole2((1, D)), _whole2((D, TW)), _whole2((1, TW)),
        ],
        out_specs=(_rows((ROWS_BLK, D)), _rows((ROWS_BLK, TW))),
        out_shape=(jax.ShapeDtypeStruct((N, D), jnp.float32),
                   jax.ShapeDtypeStruct((N, TW), jnp.float32)),
    )(h, agg_p, sat_p, a, b, dm, bias, vw, oh)


def _final(h, agg_p, sat_p, x, a, b, dm, bias, rh, rx, rb):
    return pl.pallas_call(
        _final_body, grid=(NBLK,),
        in_specs=[
            _rows((ROWS_BLK, D)),
            _whole3((NC, ROWS_BLK, TW)),
            _whole3((NC, ROWS_BLK, DE)),
            _rows((ROWS_BLK, D)),
            _whole2((D, D)), _whole2((TW, D)), _whole2((DE, D)),
            _whole2((1, D)), _whole2((D, H2)), _whole2((D, H2)),
            _whole2((1, H2)),
        ],
        out_specs=_whole2((1, H2)),
        out_shape=jax.ShapeDtypeStruct((1, H2), jnp.float32),
    )(h, agg_p, sat_p, x, a, b, dm, bias, rh, rx, rb)


def kernel(x, edge_index, edge_attr, params):
    f32 = jnp.float32
    uw, ub = params["U_w"], params["U_b"]
    vw, vb = params["V_w"], params["V_b"]
    ew, eb = params["E_w"], params["E_b"]
    rw, rb = params["R_w"], params["R_b"]

    a_w = uw[:, :D, :]
    b_mid = uw[:, D:D + H1, :]
    u_tail = uw[:, D + H1:, :]
    crow = (jnp.einsum("kh,khd->kd", vb, b_mid)
            + jnp.einsum("e,ked->kd", eb, u_tail))
    b_w = jnp.zeros((3, TW, D), f32).at[:, :H1, :].set(b_mid)
    b_w = b_w.at[:, H1, :].set(crow)
    dm_w = jnp.einsum("ef,kfd->ked", ew, u_tail)
    vp = jnp.zeros((3, D, TW), f32).at[:, :, :H1].set(vw)
    oh = jnp.zeros((1, TW), f32).at[0, H1].set(1.0)
    ub2 = ub[:, None, :]
    rh, rx = rw[:D, :], rw[D:, :]
    rb2 = rb[None, :]

    idx3 = (edge_index.astype(jnp.int32)
            .reshape(2, NCHUNK_REAL, CH).transpose(1, 0, 2))
    z64 = jnp.zeros((N_ACC, TW), f32)
    z16 = jnp.zeros((N_ACC, DE), f32)

    xv = _pre0(x, vp[0], oh)
    agg0 = _make_edge_agg()(xv, idx3, z64)
    sat = _make_attr_agg()(idx3, edge_attr, z16)
    h1, h1v = _mid(x, agg0, sat, a_w[0], b_w[0], dm_w[0], ub2[0], vp[1], oh)
    agg1 = _make_edge_agg()(h1v, idx3, z64)
    h2, h2v = _mid(h1, agg1, sat, a_w[1], b_w[1], dm_w[1], ub2[1], vp[2], oh)
    agg2 = _make_edge_agg()(h2v, idx3, z64)
    colsum = _final(h2, agg2, sat, x, a_w[2], b_w[2], dm_w[2], ub2[2],
                    rh, rx, rb2)
    readout = jnp.tanh(colsum)
    return readout @ params["lin0_w"] + params["lin0_b"]


# speedup vs baseline: 1.1404x; 1.1404x over previous
import functools

import jax
import jax.numpy as jnp
from jax import lax
from jax.experimental import pallas as pl
from jax.experimental.pallas import tpu as pltpu
from jax.experimental.pallas import tpu_sc as plsc

N = 10000
E = 320000
D = 128
H1 = 50
DE = 16
H2 = 80
TW = 64

CH = 128
NCHUNK_REAL = E // CH
NC, NS = 2, 16
NTILE = NC * NS
CPT = 80
NBUF = 4
assert (NTILE - 1) * CPT < NCHUNK_REAL <= NTILE * CPT
RPT = 632
N_ACC = RPT * NS

ROWS_BLK = 2000
NBLK = N // ROWS_BLK

_SELU_ALPHA = 1.6732632423543772
_SELU_SCALE = 1.0507009873554805


def _selu(x):
    return _SELU_SCALE * jnp.where(x > 0, x, _SELU_ALPHA * (jnp.exp(x) - 1.0))


def _mesh():
    return plsc.VectorSubcoreMesh(
        core_axis_name="c", subcore_axis_name="s",
        num_cores=NC, num_subcores=NS)


TAIL = NCHUNK_REAL - (NTILE - 1) * CPT


def _stage_idx(idx3, buf, base, full):
    @pl.when(full)
    def _():
        pltpu.sync_copy(idx3.at[pl.ds(base, CPT)], buf)

    @pl.when(jnp.logical_not(full))
    def _():
        pltpu.sync_copy(idx3.at[pl.ds(base, TAIL)], buf.at[pl.ds(0, TAIL)])


@functools.lru_cache(maxsize=None)
def _make_edge_agg():
    scratch = [
        pltpu.VMEM_SHARED((N_ACC, TW), jnp.bfloat16),
        pltpu.VMEM((CPT, 2, CH), jnp.int32),
        pltpu.VMEM((NBUF, CH, TW), jnp.bfloat16),
        pltpu.SemaphoreType.DMA((NBUF,)),
        pltpu.SemaphoreType.DMA((NBUF,)),
    ]

    def body(table, idx3, z64, out_agg, acc, idxl, rows, gsem, ssem):
        c = lax.axis_index("c")
        s = lax.axis_index("s")
        base = (c * NS + s) * CPT

        pltpu.sync_copy(z64.at[pl.ds(s * RPT, RPT)], acc.at[pl.ds(s * RPT, RPT)])

        n_real = jnp.minimum(NCHUNK_REAL - base, CPT)
        _stage_idx(idx3, idxl, base, base + CPT <= NCHUNK_REAL)

        plsc.subcore_barrier()

        def fire_g(j, slot):
            pltpu.async_copy(table.at[idxl.at[j, 0]], rows.at[slot],
                             gsem.at[slot])

        def wait_g(j, slot):
            pltpu.make_async_copy(table.at[idxl.at[j, 0]], rows.at[slot],
                                  gsem.at[slot]).wait()

        def fire_s(j, slot):
            pltpu.async_copy(rows.at[slot], acc.at[idxl.at[j, 1]],
                             ssem.at[slot], add=True)

        def wait_s(j, slot):
            pltpu.make_async_copy(rows.at[slot], acc.at[idxl.at[j, 1]],
                                  ssem.at[slot]).wait()

        for p in range(NBUF - 1):
            @pl.when(p < n_real)
            def _(p=p):
                fire_g(p, p)

        def step(j, carry):
            slot = lax.rem(j, NBUF)
            fslot = lax.rem(j + NBUF - 1, NBUF)

            @pl.when(j + NBUF - 1 < n_real)
            def _():
                @pl.when(j >= 1)
                def _():
                    wait_s(j - 1, fslot)

                fire_g(j + NBUF - 1, fslot)

            wait_g(j, slot)
            fire_s(j, slot)
            return carry

        lax.fori_loop(0, n_real, step, 0)

        def drain(k, carry):
            j = n_real - NBUF + k

            @pl.when(j >= 0)
            def _():
                wait_s(j, lax.rem(j, NBUF))
            return carry

        lax.fori_loop(0, NBUF, drain, 0)

        plsc.subcore_barrier()

        pltpu.sync_copy(acc.at[pl.ds(s * RPT, RPT)],
                        out_agg.at[c, pl.ds(s * RPT, RPT)])

    return pl.kernel(body,
                     out_type=jax.ShapeDtypeStruct((NC, N_ACC, TW),
                                                   jnp.bfloat16),
                     mesh=_mesh(), scratch_types=scratch,
                     compiler_params=pltpu.CompilerParams(
                         use_tc_tiling_on_sc=False))


@functools.lru_cache(maxsize=None)
def _make_attr_agg():
    scratch = [
        pltpu.VMEM_SHARED((N_ACC, DE), jnp.float32),
        pltpu.VMEM((CPT, 2, CH), jnp.int32),
        pltpu.VMEM((2, CH, DE), jnp.float32),
        pltpu.SemaphoreType.DMA((2,)),
        pltpu.SemaphoreType.DMA((2,)),
    ]

    def body(idx3, attr, z16, out_sattr, acc2, idxl, attrb, asem, ssem):
        c = lax.axis_index("c")
        s = lax.axis_index("s")
        base = (c * NS + s) * CPT

        pltpu.sync_copy(z16.at[pl.ds(s * RPT, RPT)],
                        acc2.at[pl.ds(s * RPT, RPT)])

        n_real = jnp.minimum(NCHUNK_REAL - base, CPT)
        _stage_idx(idx3, idxl, base, base + CPT <= NCHUNK_REAL)

        plsc.subcore_barrier()

        def fire_a(j, slot):
            pltpu.async_copy(attr.at[pl.ds((base + j) * CH, CH)],
                             attrb.at[slot], asem.at[slot])

        def wait_a(j, slot):
            pltpu.make_async_copy(attr.at[pl.ds((base + j) * CH, CH)],
                                  attrb.at[slot], asem.at[slot]).wait()

        def fire_s(j, slot):
            pltpu.async_copy(attrb.at[slot], acc2.at[idxl.at[j, 1]],
                             ssem.at[slot], add=True)

        def wait_s(j, slot):
            pltpu.make_async_copy(attrb.at[slot], acc2.at[idxl.at[j, 1]],
                                  ssem.at[slot]).wait()

        fire_a(0, 0)

        def step(j, carry):
            slot = lax.rem(j, 2)
            nslot = lax.rem(j + 1, 2)

            @pl.when(j + 1 < n_real)
            def _():
                @pl.when(j >= 1)
                def _():
                    wait_s(j - 1, nslot)

                fire_a(j + 1, nslot)

            wait_a(j, slot)
            fire_s(j, slot)
            return carry

        lax.fori_loop(0, n_real, step, 0)

        wait_s(n_real - 2, lax.rem(n_real - 2, 2))
        wait_s(n_real - 1, lax.rem(n_real - 1, 2))

        plsc.subcore_barrier()

        pltpu.sync_copy(acc2.at[pl.ds(s * RPT, RPT)],
                        out_sattr.at[c, pl.ds(s * RPT, RPT)])

    return pl.kernel(body,
                     out_type=jax.ShapeDtypeStruct((NC, N_ACC, DE),
                                                   jnp.float32),
                     mesh=_mesh(), scratch_types=scratch,
                     compiler_params=pltpu.CompilerParams(
                         use_tc_tiling_on_sc=False))


def _dot(a, b):
    return jnp.dot(a, b, preferred_element_type=jnp.float32)


def _pre0_body(x_ref, vw_ref, oh_ref, out_ref):
    out_ref[...] = (_dot(x_ref[...], vw_ref[...])
                    + oh_ref[...]).astype(jnp.bfloat16)


def _mid_body(h_ref, agg_ref, sat_ref, a_ref, b_ref, dm_ref, bias_ref,
              vw_ref, oh_ref, h_out, hv_out):
    agg = (agg_ref[0].astype(jnp.float32)
           + agg_ref[1].astype(jnp.float32))
    sat = sat_ref[0] + sat_ref[1]
    z = (_dot(h_ref[...], a_ref[...]) + _dot(agg, b_ref[...])
         + _dot(sat, dm_ref[...]) + bias_ref[...])
    h = _selu(z)
    h_out[...] = h
    hv_out[...] = (_dot(h, vw_ref[...]) + oh_ref[...]).astype(jnp.bfloat16)


def _final_body(h_ref, agg_ref, sat_ref, x_ref, a_ref, b_ref, dm_ref,
                bias_ref, rh_ref, rx_ref, rb_ref, out_ref):
    i = pl.program_id(0)
    agg = (agg_ref[0].astype(jnp.float32)
           + agg_ref[1].astype(jnp.float32))
    sat = sat_ref[0] + sat_ref[1]
    z = (_dot(h_ref[...], a_ref[...]) + _dot(agg, b_ref[...])
         + _dot(sat, dm_ref[...]) + bias_ref[...])
    h = _selu(z)
    r = _selu(_dot(h, rh_ref[...]) + _dot(x_ref[...], rx_ref[...])
              + rb_ref[...])

    @pl.when(i == 0)
    def _():
        out_ref[...] = jnp.zeros_like(out_ref)

    out_ref[...] += jnp.sum(r, axis=0, keepdims=True)


def _rows(shape):
    return pl.BlockSpec(shape, lambda i: (i, 0))


def _whole2(shape):
    return pl.BlockSpec(shape, lambda i: (0, 0))


def _whole3(shape):
    return pl.BlockSpec(shape, lambda i: (0, i, 0))


def _pre0(x, vw, oh):
    return pl.pallas_call(
        _pre0_body, grid=(NBLK,),
        in_specs=[_rows((ROWS_BLK, D)), _whole2((D, TW)), _whole2((1, TW))],
        out_specs=_rows((ROWS_BLK, TW)),
        out_shape=jax.ShapeDtypeStruct((N, TW), jnp.bfloat16),
    )(x, vw, oh)


def _mid(h, agg_p, sat_p, a, b, dm, bias, vw, oh):
    return pl.pallas_call(
        _mid_body, grid=(NBLK,),
        in_specs=[
            _rows((ROWS_BLK, D)),
            _whole3((NC, ROWS_BLK, TW)),
            _whole3((NC, ROWS_BLK, DE)),
            _whole2((D, D)), _whole2((TW, D)), _whole2((DE, D)),
            _whole2((1, D)), _whole2((D, TW)), _whole2((1, TW)),
        ],
        out_specs=(_rows((ROWS_BLK, D)), _rows((ROWS_BLK, TW))),
        out_shape=(jax.ShapeDtypeStruct((N, D), jnp.float32),
                   jax.ShapeDtypeStruct((N, TW), jnp.bfloat16)),
    )(h, agg_p, sat_p, a, b, dm, bias, vw, oh)


def _final(h, agg_p, sat_p, x, a, b, dm, bias, rh, rx, rb):
    return pl.pallas_call(
        _final_body, grid=(NBLK,),
        in_specs=[
            _rows((ROWS_BLK, D)),
            _whole3((NC, ROWS_BLK, TW)),
            _whole3((NC, ROWS_BLK, DE)),
            _rows((ROWS_BLK, D)),
            _whole2((D, D)), _whole2((TW, D)), _whole2((DE, D)),
            _whole2((1, D)), _whole2((D, H2)), _whole2((D, H2)),
            _whole2((1, H2)),
        ],
        out_specs=_whole2((1, H2)),
        out_shape=jax.ShapeDtypeStruct((1, H2), jnp.float32),
    )(h, agg_p, sat_p, x, a, b, dm, bias, rh, rx, rb)


def kernel(x, edge_index, edge_attr, params):
    f32 = jnp.float32
    uw, ub = params["U_w"], params["U_b"]
    vw, vb = params["V_w"], params["V_b"]
    ew, eb = params["E_w"], params["E_b"]
    rw, rb = params["R_w"], params["R_b"]

    a_w = uw[:, :D, :]
    b_mid = uw[:, D:D + H1, :]
    u_tail = uw[:, D + H1:, :]
    crow = (jnp.einsum("kh,khd->kd", vb, b_mid)
            + jnp.einsum("e,ked->kd", eb, u_tail))
    b_w = jnp.zeros((3, TW, D), f32).at[:, :H1, :].set(b_mid)
    b_w = b_w.at[:, H1, :].set(crow)
    dm_w = jnp.einsum("ef,kfd->ked", ew, u_tail)
    vp = jnp.zeros((3, D, TW), f32).at[:, :, :H1].set(vw)
    oh = jnp.zeros((1, TW), f32).at[0, H1].set(1.0)
    ub2 = ub[:, None, :]
    rh, rx = rw[:D, :], rw[D:, :]
    rb2 = rb[None, :]

    idx3 = (edge_index.astype(jnp.int32)
            .reshape(2, NCHUNK_REAL, CH).transpose(1, 0, 2))
    z64 = jnp.zeros((N_ACC, TW), jnp.bfloat16)
    z16 = jnp.zeros((N_ACC, DE), f32)

    xv = _pre0(x, vp[0], oh)
    agg0 = _make_edge_agg()(xv, idx3, z64)
    sat = _make_attr_agg()(idx3, edge_attr, z16)
    h1, h1v = _mid(x, agg0, sat, a_w[0], b_w[0], dm_w[0], ub2[0], vp[1], oh)
    agg1 = _make_edge_agg()(h1v, idx3, z64)
    h2, h2v = _mid(h1, agg1, sat, a_w[1], b_w[1], dm_w[1], ub2[1], vp[2], oh)
    agg2 = _make_edge_agg()(h2v, idx3, z64)
    colsum = _final(h2, agg2, sat, x, a_w[2], b_w[2], dm_w[2], ub2[2],
                    rh, rx, rb2)
    readout = jnp.tanh(colsum)
    return readout @ params["lin0_w"] + params["lin0_b"]
